# Initial kernel scaffold; baseline (speedup 1.0000x reference)
import jax
import jax.numpy as jnp
from jax.experimental import pallas as pl


def kernel(efeat, nfeat, edge_index, W_e, W_s, W_d, b1, W_out, b_out, gamma, beta):
    raise NotImplementedError("write your pallas kernel here")


# speedup vs baseline: 2.9238x; 2.9238x over previous
import jax
import jax.numpy as jnp
from jax import lax
from jax.experimental import pallas as pl
from jax.experimental.pallas import tpu as pltpu
from jax.experimental.pallas import tpu_sc as plsc

N_NODES = 10000
N_EDGES = 320000
D = 128
H = 128


_NB = 2000


def _proj_body(nf_ref, ws_ref, wd_ref, ps_ref, pd_ref):
    x = nf_ref[...]
    ps_ref[...] = jnp.dot(x, ws_ref[...], preferred_element_type=jnp.float32)
    pd_ref[...] = jnp.dot(x, wd_ref[...], preferred_element_type=jnp.float32)


def _project_nodes(nfeat, W_s, W_d):
    return pl.pallas_call(
        _proj_body,
        grid=(N_NODES // _NB,),
        in_specs=[
            pl.BlockSpec((_NB, D), lambda i: (i, 0)),
            pl.BlockSpec((D, H), lambda i: (0, 0)),
            pl.BlockSpec((D, H), lambda i: (0, 0)),
        ],
        out_specs=[
            pl.BlockSpec((_NB, H), lambda i: (i, 0)),
            pl.BlockSpec((_NB, H), lambda i: (i, 0)),
        ],
        out_shape=[
            jax.ShapeDtypeStruct((N_NODES, H), jnp.float32),
            jax.ShapeDtypeStruct((N_NODES, H), jnp.float32),
        ],
    )(nfeat, W_s, W_d)


_NC = 2
_NS = 16
_NW = _NC * _NS
_C = 128
_NCHUNK = N_EDGES // _C
_TPW = -(-_NCHUNK // _NW)


def _gather_body(src_hbm, dst_hbm, ps_hbm, pd_hbm, out_hbm,
                 isrc, idst, ra, rb, sem):
    wid = lax.axis_index("s") * _NC + lax.axis_index("c")

    def step(t, carry):
        g = wid + _NW * t

        @pl.when(g < _NCHUNK)
        def _():
            off = g * _C
            pltpu.sync_copy(src_hbm.at[pl.ds(off, _C)], isrc)
            pltpu.sync_copy(dst_hbm.at[pl.ds(off, _C)], idst)
            cp1 = pltpu.async_copy(ps_hbm.at[isrc], ra, sem)
            cp2 = pltpu.async_copy(pd_hbm.at[idst], rb, sem)
            cp1.wait()
            cp2.wait()

            def add_row(e, c2):
                for j in range(H // 16):
                    sl = pl.ds(j * 16, 16)
                    ra[e, sl] = ra[e, sl] + rb[e, sl]
                return c2

            lax.fori_loop(0, _C, add_row, 0)
            pltpu.sync_copy(ra, out_hbm.at[pl.ds(off, _C)])

        return carry

    lax.fori_loop(0, _TPW, step, 0)


def _gather_sum(src, dst, ps, pd):
    mesh = plsc.VectorSubcoreMesh(core_axis_name="c", subcore_axis_name="s")
    f = pl.kernel(
        _gather_body,
        mesh=mesh,
        out_type=jax.ShapeDtypeStruct((N_EDGES, H), jnp.float32),
        scratch_types=[
            pltpu.VMEM((_C,), jnp.int32),
            pltpu.VMEM((_C,), jnp.int32),
            pltpu.VMEM((_C, H), jnp.float32),
            pltpu.VMEM((_C, H), jnp.float32),
            pltpu.SemaphoreType.DMA,
        ],
    )
    return f(src, dst, ps, pd)


_EB = 2000


def _edge_body(ef_ref, g_ref, we_ref, wo_ref, b1_ref, bo_ref, gm_ref, bt_ref,
               out_ref):
    ef = ef_ref[...]
    pre = (jnp.dot(ef, we_ref[...], preferred_element_type=jnp.float32)
           + g_ref[...] + b1_ref[...])
    h = pre * (1.0 / (1.0 + jnp.exp(-pre)))
    o = jnp.dot(h, wo_ref[...], preferred_element_type=jnp.float32) + bo_ref[...]
    mean = jnp.mean(o, axis=-1, keepdims=True)
    cent = o - mean
    var = jnp.mean(cent * cent, axis=-1, keepdims=True)
    out_ref[...] = (gm_ref[...] * cent * lax.rsqrt(var + 1e-5)
                    + bt_ref[...] + ef)


def _edge_mlp(efeat, gsum, W_e, W_out, b1, b_out, gamma, beta):
    vec = lambda: pl.BlockSpec((1, D), lambda i: (0, 0))
    return pl.pallas_call(
        _edge_body,
        grid=(N_EDGES // _EB,),
        in_specs=[
            pl.BlockSpec((_EB, D), lambda i: (i, 0)),
            pl.BlockSpec((_EB, H), lambda i: (i, 0)),
            pl.BlockSpec((D, H), lambda i: (0, 0)),
            pl.BlockSpec((H, D), lambda i: (0, 0)),
            vec(), vec(), vec(), vec(),
        ],
        out_specs=pl.BlockSpec((_EB, D), lambda i: (i, 0)),
        out_shape=jax.ShapeDtypeStruct((N_EDGES, D), jnp.float32),
    )(efeat, gsum, W_e, W_out,
      b1.reshape(1, D), b_out.reshape(1, D),
      gamma.reshape(1, D), beta.reshape(1, D))


def kernel(efeat, nfeat, edge_index, W_e, W_s, W_d, b1, W_out, b_out, gamma,
           beta):
    src = edge_index[0].astype(jnp.int32)
    dst = edge_index[1].astype(jnp.int32)
    ps, pd = _project_nodes(nfeat, W_s, W_d)
    gsum = _gather_sum(src, dst, ps, pd)
    out = _edge_mlp(efeat, gsum, W_e, W_out, b1, b_out, gamma, beta)
    return (out, nfeat)
